# baseline (speedup 1.0000x reference)
import functools
import math

import jax
import jax.numpy as jnp
from jax import lax
from jax.experimental import pallas as pl
from jax.experimental.pallas import tpu as pltpu
from jax.experimental.pallas import tpu_sc as plsc

_NROW = 4096
_NCOL = 200
_D = 32
_NW = 32
_BW = _NROW // _NW
_GT = _D // 8
_NG = _NCOL // 8
_SCALE = math.sqrt(float(_D))

_mesh = plsc.VectorSubcoreMesh(core_axis_name="c", subcore_axis_name="s")


_NCH = 7812


@functools.partial(
    pl.kernel,
    out_type=jax.ShapeDtypeStruct((32000000,), jnp.float32),
    mesh=_mesh,
    compiler_params=pltpu.CompilerParams(
        use_tc_tiling_on_sc=True, needs_layout_passes=False
    ),
    scratch_types=[
        pltpu.VMEM((_D, 128), jnp.float32),
        pltpu.VMEM((_D, 128), jnp.float32),
        pltpu.VMEM((128, 33), jnp.float32),
        pltpu.VMEM((128, 33), jnp.float32),
        pltpu.VMEM((4096,), jnp.float32),
        pltpu.VMEM((4096,), jnp.float32),
        pltpu.VMEM((_D, 64), jnp.float32),
        pltpu.SemaphoreType.DMA,
        pltpu.SemaphoreType.DMA,
        pltpu.SemaphoreType.DMA,
        pltpu.SemaphoreType.DMA,
    ],
)
def _table_relayout(tablet_hbm, out_hbm, buf0, buf1, pad0, pad1,
                    lin0, lin1, tail_v, rs0, rs1, ws0, ws1):
    wid = lax.axis_index("s") * 2 + lax.axis_index("c")
    rsem = (rs0, rs1)
    wsem = (ws0, ws1)
    bufs = (buf0, buf1)
    pads = (pad0, pad1)
    lins = (lin0, lin1)
    lane = jax.lax.iota(jnp.int32, 16)

    def rd(c, b):
        return pltpu.make_async_copy(
            tablet_hbm.at[:, pl.ds(128 * c, 128)], bufs[b], rsem[b]
        )

    def wr(c, b):
        return pltpu.make_async_copy(
            lins[b], out_hbm.at[pl.ds(4096 * c, 4096)], wsem[b]
        )

    def cid(i):
        return wid + 32 * i

    def xform(b):
        buf = bufs[b]
        pad = pads[b]
        lin = lins[b]
        for k in range(8):
            w_idx = lane + 16 * k
            for d in range(_D):
                v = buf[d, pl.ds(16 * k, 16)]
                plsc.store_scatter(pad, [w_idx, jax.lax.broadcast(d, (16,))],
                                   v * _SCALE)

        @plsc.parallel_loop(0, 128, unroll=4)
        def _(r):
            for h in range(2):
                lin[pl.ds(r * _D + 16 * h, 16)] = pad[r, pl.ds(16 * h, 16)]

    def step(i, b, fire, wait_w):
        rd(cid(i), b).wait()
        if wait_w:
            wr(cid(i - 2), b).wait()
        xform(b)
        if fire:
            rd(cid(i + 2), b).start()
        wr(cid(i), b).start()

    rd(cid(0), 0).start()
    rd(cid(1), 1).start()
    step(0, 0, True, False)
    step(1, 1, True, False)

    def body(k, carry):
        i = 2 * k
        step(i, 0, True, True)
        step(i + 1, 1, True, True)
        return carry

    lax.fori_loop(1, 121, body, 0)

    step(242, 0, False, True)
    step(243, 1, False, True)
    wr(cid(242), 0).wait()
    wr(cid(243), 1).wait()

    @pl.when(wid < 4)
    def _():
        pltpu.sync_copy(tablet_hbm.at[:, pl.ds(128 * cid(244), 128)], buf0)
        xform(0)
        pltpu.sync_copy(lin0, out_hbm.at[pl.ds(4096 * cid(244), 4096)])

    @pl.when(wid == 1)
    def _():
        pltpu.sync_copy(tablet_hbm.at[:, pl.ds(128 * _NCH, 64)], tail_v)
        pad = pad0
        lin = lin0
        for k in range(4):
            w_idx = lane + 16 * k
            for d in range(_D):
                v = tail_v[d, pl.ds(16 * k, 16)]
                plsc.store_scatter(pad, [w_idx, jax.lax.broadcast(d, (16,))],
                                   v * _SCALE)
        for r in range(64):
            for h in range(2):
                lin[pl.ds(r * _D + 16 * h, 16)] = pad[r, pl.ds(16 * h, 16)]
        pltpu.sync_copy(lin0.at[pl.ds(0, 2048)],
                        out_hbm.at[pl.ds(4096 * _NCH, 2048)])


@functools.partial(
    pl.kernel,
    out_type=jax.ShapeDtypeStruct((_NCOL, _GT, _NW, 8, 128), jnp.float32),
    mesh=_mesh,
    compiler_params=pltpu.CompilerParams(
        use_tc_tiling_on_sc=False, needs_layout_passes=False
    ),
    scratch_types=[
        pltpu.VMEM((_NG, 8, _BW), jnp.int32),
        pltpu.VMEM((8, _BW, _D), jnp.float32),
        pltpu.VMEM((8, _D, 129), jnp.float32),
    ] + [pltpu.SemaphoreType.DMA] * 16,
)
def _emb_lookup(tokens_hbm, table_hbm, out_hbm, idx_v, rows_v, tr_v,
                *sems):
    gsem = sems[:8]
    wsem = sems[8:]
    wid = lax.axis_index("s") * 2 + lax.axis_index("c")
    pltpu.sync_copy(tokens_hbm.at[:, wid], idx_v)

    lane = jax.lax.iota(jnp.int32, 16)
    half_dim = [lane + (16 * h) for h in range(2)]

    def gather_desc(t, jc, s):
        return pltpu.make_async_copy(
            table_hbm.at[idx_v.at[t, jc]],
            rows_v.at[s],
            gsem[s],
        )

    def write_descs(t, jc):
        return [
            pltpu.make_async_copy(
                tr_v.at[jc, pl.ds(g * 8, 8), pl.ds(0, 128)],
                out_hbm.at[8 * t + jc, g, wid],
                wsem[jc],
            )
            for g in range(_GT)
        ]

    def do_group(t, fire_next, wait_prev_write):
        for jc in range(8):
            gather_desc(t, jc, jc).wait()
            if wait_prev_write:
                for d in write_descs(t - 1, jc):
                    d.wait()
            buf = rows_v.at[jc]
            dst = tr_v.at[jc]

            @plsc.parallel_loop(0, _BW, unroll=8)
            def _(r):
                rv = jax.lax.broadcast(r, (16,))
                for h in range(2):
                    v = buf[r, pl.ds(16 * h, 16)]
                    plsc.store_scatter(dst, [half_dim[h], rv], v)
            if fire_next:
                gather_desc(t + 1, jc, jc).start()
            for d in write_descs(t, jc):
                d.start()

    for jc in range(8):
        gather_desc(0, jc, jc).start()
    do_group(0, True, False)

    def body(t, carry):
        do_group(t, True, True)
        return carry

    lax.fori_loop(1, _NG - 1, body, 0)

    do_group(_NG - 1, False, True)
    for jc in range(8):
        for d in write_descs(_NG - 1, jc):
            d.wait()


def kernel(tokens, table):
    tok4 = tokens.T.astype(jnp.int32).reshape(_NG, 8, _NW, 128)
    tl = _table_relayout(table.T)
    kout = _emb_lookup(tok4.transpose(0, 2, 1, 3), tl.reshape(1000000, _D))
    return kout.transpose(2, 4, 0, 1, 3).reshape(_NROW, _NCOL, _D)


# speedup vs baseline: 1.4505x; 1.4505x over previous
import functools
import math

import jax
import jax.numpy as jnp
from jax import lax
from jax.experimental import pallas as pl
from jax.experimental.pallas import tpu as pltpu
from jax.experimental.pallas import tpu_sc as plsc

_NROW = 4096
_NCOL = 200
_D = 32
_NW = 32
_BW = _NROW // _NW
_GT = _D // 8
_NG = _NCOL // 8
_SCALE = math.sqrt(float(_D))

_mesh = plsc.VectorSubcoreMesh(core_axis_name="c", subcore_axis_name="s")


@functools.partial(
    pl.kernel,
    out_type=jax.ShapeDtypeStruct((_NCOL, _GT, _NW, 8, 128), jnp.float32),
    mesh=_mesh,
    compiler_params=pltpu.CompilerParams(
        use_tc_tiling_on_sc=False, needs_layout_passes=False
    ),
    scratch_types=[
        pltpu.VMEM((_NG, 8, _BW), jnp.int32),
        pltpu.VMEM((8, _BW, _D), jnp.float32),
        pltpu.VMEM((8, _D, 129), jnp.float32),
    ] + [pltpu.SemaphoreType.DMA] * 16,
)
def _emb_lookup(tokens_hbm, table_hbm, out_hbm, idx_v, rows_v, tr_v,
                *sems):
    gsem = sems[:8]
    wsem = sems[8:]
    wid = lax.axis_index("s") * 2 + lax.axis_index("c")
    pltpu.sync_copy(tokens_hbm.at[:, wid], idx_v)

    lane = jax.lax.iota(jnp.int32, 16)
    half_dim = [lane + (16 * h) for h in range(2)]

    def gather_desc(t, jc, s):
        return pltpu.make_async_copy(
            table_hbm.at[idx_v.at[t, jc]],
            rows_v.at[s],
            gsem[s],
        )

    def write_descs(t, jc):
        return [
            pltpu.make_async_copy(
                tr_v.at[jc, pl.ds(g * 8, 8), pl.ds(0, 128)],
                out_hbm.at[8 * t + jc, g, wid],
                wsem[jc],
            )
            for g in range(_GT)
        ]

    def do_group(t, fire_next, wait_prev_write):
        for jc in range(8):
            gather_desc(t, jc, jc).wait()
            if wait_prev_write:
                for d in write_descs(t - 1, jc):
                    d.wait()
            buf = rows_v.at[jc]
            dst = tr_v.at[jc]

            @plsc.parallel_loop(0, _BW, unroll=8)
            def _(r):
                rv = jax.lax.broadcast(r, (16,))
                for h in range(2):
                    v = buf[r, pl.ds(16 * h, 16)]
                    plsc.store_scatter(dst, [half_dim[h], rv], v * _SCALE)
            if fire_next:
                gather_desc(t + 1, jc, jc).start()
            for d in write_descs(t, jc):
                d.start()

    for jc in range(8):
        gather_desc(0, jc, jc).start()
    do_group(0, True, False)

    def body(t, carry):
        do_group(t, True, True)
        return carry

    lax.fori_loop(1, _NG - 1, body, 0)

    do_group(_NG - 1, False, True)
    for jc in range(8):
        for d in write_descs(_NG - 1, jc):
            d.wait()


def kernel(tokens, table):
    tok4 = tokens.T.astype(jnp.int32).reshape(_NG, 8, _NW, 128)
    kout = _emb_lookup(tok4.transpose(0, 2, 1, 3), table)
    return kout.transpose(2, 4, 0, 1, 3).reshape(_NROW, _NCOL, _D)
